# baseline (speedup 1.0000x reference)
import functools

import jax
import jax.numpy as jnp
from jax import lax
from jax.experimental import pallas as pl
from jax.experimental.pallas import tpu as pltpu
from jax.experimental.pallas import tpu_sc as plsc

N = 8192
D = 2048
FF = 1408
E = 8
BLK = 256
TB = 512
NPAD = 2 * N + E * BLK
NBLK = NPAD // BLK

NC, NS = 2, 16
NW = NC * NS
TPW = N // NW
CH_C = 16
NCH_C = TPW // CH_C
CH_E = 8
NCH_E = TPW // CH_E

_bf16 = jnp.bfloat16

_f32 = jnp.float32
_i32 = jnp.int32


def _dot_t(a, b):
    return lax.dot_general(a, b, (((1,), (1,)), ((), ())),
                           preferred_element_type=_f32)


def _silu(v):
    return v * (1.0 / (1.0 + jnp.exp(-v)))


def router_body(x_ref, rw_ref, i0_ref, i1_ref, w0_ref, w1_ref):
    l = _dot_t(rw_ref[...], x_ref[...])
    ei = lax.broadcasted_iota(_i32, (E, TB), 0)
    m0 = jnp.max(l, axis=0, keepdims=True)
    i0 = jnp.min(jnp.where(l == m0, ei, E), axis=0, keepdims=True)
    lm = jnp.where(ei == i0, -jnp.inf, l)
    m1 = jnp.max(lm, axis=0, keepdims=True)
    i1 = jnp.min(jnp.where(lm == m1, ei, E), axis=0, keepdims=True)
    w0 = 1.0 / (1.0 + jnp.exp(m1 - m0))
    w1 = 1.0 / (1.0 + jnp.exp(m0 - m1))
    i0_ref[...] = i0.reshape(1, 1, TB)
    i1_ref[...] = i1.reshape(1, 1, TB)
    w0_ref[...] = w0.reshape(1, 1, TB)
    w1_ref[...] = w1.reshape(1, 1, TB)


def _router_call(x2d, router_w):
    nb = N // TB
    o3 = jax.ShapeDtypeStruct((nb, 1, TB), _i32)
    o3f = jax.ShapeDtypeStruct((nb, 1, TB), _f32)
    return pl.pallas_call(
        router_body,
        grid=(nb,),
        in_specs=[
            pl.BlockSpec((TB, D), lambda b: (b, 0)),
            pl.BlockSpec((E, D), lambda b: (0, 0)),
        ],
        out_specs=[
            pl.BlockSpec((1, 1, TB), lambda b: (b, 0, 0)),
            pl.BlockSpec((1, 1, TB), lambda b: (b, 0, 0)),
            pl.BlockSpec((1, 1, TB), lambda b: (b, 0, 0)),
            pl.BlockSpec((1, 1, TB), lambda b: (b, 0, 0)),
        ],
        out_shape=[o3, o3, o3f, o3f],
    )(x2d, router_w)


def dispatch_body(i0_ref, i1_ref, d0_ref, d1_ref, be_ref):
    nb = N // TB
    i0 = i0_ref[...].reshape(nb, TB)
    i1 = i1_ref[...].reshape(nb, TB)

    ra = lax.broadcasted_iota(_i32, (TB, TB), 0)
    ca = lax.broadcasted_iota(_i32, (TB, TB), 1)
    U = (ra <= ca).astype(_f32)
    rb = lax.broadcasted_iota(_i32, (nb, nb), 0)
    cb = lax.broadcasted_iota(_i32, (nb, nb), 1)
    Ls = (rb > cb).astype(_f32)

    def excl_scan(m):
        rowcum = jnp.dot(m, U, preferred_element_type=_f32)
        rowtot = jnp.sum(m, axis=1, keepdims=True)
        rowoff = jnp.dot(Ls, rowtot, preferred_element_type=_f32)
        return rowcum - m + rowoff

    masks0 = [(i0 == e).astype(_f32) for e in range(E)]
    masks1 = [(i1 == e).astype(_f32) for e in range(E)]
    cnt0 = [jnp.sum(m).astype(_i32) for m in masks0]
    cnt1 = [jnp.sum(m).astype(_i32) for m in masks1]

    starts = []
    s = jnp.int32(0)
    for e in range(E):
        starts.append(s)
        seg = ((cnt0[e] + cnt1[e] + (BLK - 1)) // BLK) * BLK
        s = s + seg

    dest0 = jnp.zeros((nb, TB), _f32)
    dest1 = jnp.zeros((nb, TB), _f32)
    for e in range(E):
        r0 = excl_scan(masks0[e])
        r1 = excl_scan(masks1[e])
        base_e = starts[e].astype(_f32)
        dest0 = dest0 + masks0[e] * (base_e + r0)
        dest1 = dest1 + masks1[e] * (base_e + cnt0[e].astype(_f32) + r1)

    d0_ref[...] = dest0.astype(_i32).reshape(nb, 1, TB)
    d1_ref[...] = dest1.astype(_i32).reshape(nb, 1, TB)

    bvec = lax.broadcasted_iota(_i32, (8, 128), 1)
    acc = jnp.zeros((8, 128), _i32)
    for e in range(1, E):
        acc = acc + jnp.where(bvec >= starts[e] // BLK, 1, 0).astype(_i32)
    be_ref[...] = acc


def _dispatch_call(i0_3, i1_3):
    nb = N // TB
    return pl.pallas_call(
        dispatch_body,
        grid=(1,),
        in_specs=[
            pl.BlockSpec((nb, 1, TB), lambda i: (0, 0, 0)),
            pl.BlockSpec((nb, 1, TB), lambda i: (0, 0, 0)),
        ],
        out_specs=[
            pl.BlockSpec((nb, 1, TB), lambda i: (0, 0, 0)),
            pl.BlockSpec((nb, 1, TB), lambda i: (0, 0, 0)),
            pl.BlockSpec((8, 128), lambda i: (0, 0)),
        ],
        out_shape=[
            jax.ShapeDtypeStruct((nb, 1, TB), _i32),
            jax.ShapeDtypeStruct((nb, 1, TB), _i32),
            jax.ShapeDtypeStruct((8, 128), _i32),
        ],
    )(i0_3, i1_3)


@functools.lru_cache(maxsize=None)
def _scatter_dispatch_kernel():
    mesh = plsc.VectorSubcoreMesh(core_axis_name="c", subcore_axis_name="s")
    npair = NCH_C // 2

    @functools.partial(
        pl.kernel,
        out_type=(
            jax.ShapeDtypeStruct((NPAD, D), _f32),
            jax.ShapeDtypeStruct((NPAD,), _f32),
        ),
        mesh=mesh,
        scratch_types=[
            pltpu.VMEM((CH_C, D), _f32),
            pltpu.VMEM((CH_C, D), _f32),
            pltpu.VMEM((NCH_C, CH_C), _i32),
            pltpu.VMEM((NCH_C, CH_C), _i32),
            pltpu.VMEM((NCH_C, CH_C), _f32),
            pltpu.VMEM((NCH_C, CH_C), _f32),
            pltpu.SemaphoreType.DMA,
            pltpu.SemaphoreType.DMA,
            pltpu.SemaphoreType.DMA,
            pltpu.SemaphoreType.DMA,
        ],
    )
    def scatter_dispatch(x_hbm, d0_hbm, d1_hbm, w0_hbm, w1_hbm,
                         xg_hbm, ws_hbm, xvA, xvB, d0v, d1v, w0v, w1v,
                         semLA, semLB, semSA, semSB):
        wid = lax.axis_index("s") * NC + lax.axis_index("c")
        base = wid * TPW

        pltpu.sync_copy(d0_hbm.at[wid], d0v)
        pltpu.sync_copy(d1_hbm.at[wid], d1v)
        pltpu.sync_copy(w0_hbm.at[wid], w0v)
        pltpu.sync_copy(w1_hbm.at[wid], w1v)

        def load(c, xv, sem):
            pltpu.async_copy(x_hbm.at[pl.ds(base + c * CH_C, CH_C)], xv, sem)

        def wait_load(xv, sem):
            pltpu.make_async_copy(x_hbm.at[pl.ds(base, CH_C)], xv, sem).wait()

        def scatters(c, xv, sem):
            pltpu.async_copy(xv, xg_hbm.at[d0v.at[c]], sem)
            pltpu.async_copy(xv, xg_hbm.at[d1v.at[c]], sem)
            pltpu.async_copy(w0v.at[c], ws_hbm.at[d0v.at[c]], sem)
            pltpu.async_copy(w1v.at[c], ws_hbm.at[d1v.at[c]], sem)

        def wait_scatters(c, xv, sem):
            pltpu.make_async_copy(xv, xg_hbm.at[d0v.at[c]], sem).wait()
            pltpu.make_async_copy(xv, xg_hbm.at[d1v.at[c]], sem).wait()
            pltpu.make_async_copy(w0v.at[c], ws_hbm.at[d0v.at[c]], sem).wait()
            pltpu.make_async_copy(w1v.at[c], ws_hbm.at[d1v.at[c]], sem).wait()

        load(0, xvA, semLA)
        load(1, xvB, semLB)

        def pair(g, carry):
            a, b = 2 * g, 2 * g + 1
            wait_load(xvA, semLA)
            scatters(a, xvA, semSA)
            wait_load(xvB, semLB)
            scatters(b, xvB, semSB)
            wait_scatters(a, xvA, semSA)
            pl.when(g < npair - 1)(lambda: load(a + 2, xvA, semLA))
            wait_scatters(b, xvB, semSB)
            pl.when(g < npair - 1)(lambda: load(b + 2, xvB, semLB))
            return carry

        lax.fori_loop(0, npair, pair, 0)

    return scatter_dispatch


def group_gemm_body(be_ref, xg_ref, w1_ref, w2_ref, ws_ref, o_ref):
    h = _silu(_dot_t(xg_ref[...].astype(_bf16), w1_ref[0]))
    h = h * ws_ref[...]
    o_ref[...] = _dot_t(h.astype(_bf16), w2_ref[0])


def _group_gemm_call(be, xg, ew1, ew2, ws2):
    grid_spec = pltpu.PrefetchScalarGridSpec(
        num_scalar_prefetch=1,
        grid=(NBLK,),
        in_specs=[
            pl.BlockSpec((BLK, D), lambda b, be_r: (b, 0)),
            pl.BlockSpec((1, FF, D), lambda b, be_r: (be_r[b], 0, 0)),
            pl.BlockSpec((1, D, FF), lambda b, be_r: (be_r[b], 0, 0)),
            pl.BlockSpec((BLK, 1), lambda b, be_r: (b, 0)),
        ],
        out_specs=pl.BlockSpec((BLK, D), lambda b, be_r: (b, 0)),
    )
    return pl.pallas_call(
        group_gemm_body,
        grid_spec=grid_spec,
        out_shape=jax.ShapeDtypeStruct((NPAD, D), _f32),
    )(be, xg, ew1, ew2, ws2)


def shared_ffn_body(x_ref, w1_ref, w2_ref, rs_ref, o_ref):
    h = _silu(_dot_t(x_ref[...].astype(_bf16), w1_ref[...]))
    o_ref[...] = _dot_t(h.astype(_bf16), w2_ref[...]) + rs_ref[...]


def _shared_ffn_call(x2d, sw1, sw2, rsum):
    return pl.pallas_call(
        shared_ffn_body,
        grid=(N // BLK,),
        in_specs=[
            pl.BlockSpec((BLK, D), lambda b: (b, 0)),
            pl.BlockSpec((FF, D), lambda b: (0, 0)),
            pl.BlockSpec((D, FF), lambda b: (0, 0)),
            pl.BlockSpec((BLK, D), lambda b: (b, 0)),
        ],
        out_specs=pl.BlockSpec((BLK, D), lambda b: (b, 0)),
        out_shape=jax.ShapeDtypeStruct((N, D), _f32),
    )(x2d, sw1, sw2, rsum)


@functools.lru_cache(maxsize=None)
def _routed_sum_kernel():
    mesh = plsc.VectorSubcoreMesh(core_axis_name="c", subcore_axis_name="s")
    npair = NCH_E // 2

    @functools.partial(
        pl.kernel,
        out_type=jax.ShapeDtypeStruct((N, D), _f32),
        mesh=mesh,
        scratch_types=[
            pltpu.VMEM((NCH_E, CH_E), _i32),
            pltpu.VMEM((NCH_E, CH_E), _i32),
            pltpu.VMEM((CH_E, D), _f32),
            pltpu.VMEM((CH_E, D), _f32),
            pltpu.VMEM((CH_E, D), _f32),
            pltpu.VMEM((CH_E, D), _f32),
            pltpu.SemaphoreType.DMA,
            pltpu.SemaphoreType.DMA,
            pltpu.SemaphoreType.DMA,
            pltpu.SemaphoreType.DMA,
        ],
    )
    def routed_sum(rows_hbm, d0_hbm, d1_hbm, out_hbm,
                   p0v, p1v, accA, r1A, accB, r1B,
                   semA, semB, semStA, semStB):
        wid = lax.axis_index("s") * NC + lax.axis_index("c")
        base = wid * TPW

        pltpu.sync_copy(d0_hbm.at[wid], p0v)
        pltpu.sync_copy(d1_hbm.at[wid], p1v)

        def gathers(c, acc, r1, sem):
            pltpu.async_copy(rows_hbm.at[p0v.at[c]], acc, sem)
            pltpu.async_copy(rows_hbm.at[p1v.at[c]], r1, sem)

        def wait_gathers(c, acc, r1, sem):
            pltpu.make_async_copy(rows_hbm.at[p0v.at[c]], acc, sem).wait()
            pltpu.make_async_copy(rows_hbm.at[p1v.at[c]], r1, sem).wait()

        def compute(acc, r1):
            for r in range(CH_E):
                def col(j, jc):
                    sl = pl.ds(j * 16, 16)
                    acc[r, sl] = acc[r, sl] + r1[r, sl]
                    return jc
                lax.fori_loop(0, D // 16, col, 0, unroll=8)

        def store(c, acc, sem):
            pltpu.async_copy(acc, out_hbm.at[pl.ds(base + c * CH_E, CH_E)], sem)

        def wait_store(acc, sem):
            pltpu.make_async_copy(
                acc, out_hbm.at[pl.ds(base, CH_E)], sem).wait()

        gathers(0, accA, r1A, semA)

        def pair(g, carry):
            a, b = 2 * g, 2 * g + 1
            wait_gathers(a, accA, r1A, semA)
            gathers(b, accB, r1B, semB)
            compute(accA, r1A)
            store(a, accA, semStA)
            wait_gathers(b, accB, r1B, semB)
            wait_store(accA, semStA)
            pl.when(g < npair - 1)(lambda: gathers(a + 2, accA, r1A, semA))
            compute(accB, r1B)
            store(b, accB, semStB)
            wait_store(accB, semStB)
            return carry

        lax.fori_loop(0, npair, pair, 0)

    return routed_sum


def kernel(x, shared_w1, shared_w2, expert_w1, expert_w2, router_w):
    bs, seq, d = x.shape
    x2d = x.reshape(N, D)

    i0_3, i1_3, w0_3, w1_3 = _router_call(x2d, router_w)
    d0_3, d1_3, be_full = _dispatch_call(i0_3, i1_3)

    ew1b = expert_w1.astype(_bf16)
    ew2b = expert_w2.astype(_bf16)
    sw1b = shared_w1[0].astype(_bf16)
    sw2b = shared_w2[0].astype(_bf16)

    d0c = d0_3.reshape(NW, NCH_C, CH_C)
    d1c = d1_3.reshape(NW, NCH_C, CH_C)
    w0c = w0_3.reshape(NW, NCH_C, CH_C)
    w1c = w1_3.reshape(NW, NCH_C, CH_C)
    d0e = d0_3.reshape(NW, NCH_E, CH_E)
    d1e = d1_3.reshape(NW, NCH_E, CH_E)
    be = be_full[0, :NBLK]

    xg, ws = _scatter_dispatch_kernel()(x2d, d0c, d1c, w0c, w1c)
    rows = _group_gemm_call(be, xg, ew1b, ew2b, ws.reshape(NPAD, 1))
    rsum = _routed_sum_kernel()(rows, d0e, d1e)
    out2d = _shared_ffn_call(x2d, sw1b, sw2b, rsum)

    return (out2d.reshape(bs, seq, d), jnp.float32(0.0))


# speedup vs baseline: 1.1394x; 1.1394x over previous
import functools

import jax
import jax.numpy as jnp
from jax import lax
from jax.experimental import pallas as pl
from jax.experimental.pallas import tpu as pltpu
from jax.experimental.pallas import tpu_sc as plsc

N = 8192
D = 2048
FF = 1408
E = 8
BLK = 256
TB = 512
NPAD = 2 * N + E * BLK
NBLK = NPAD // BLK

NC, NS = 2, 16
NW = NC * NS
TPW = N // NW
CH_C = 16
NCH_C = TPW // CH_C
CH_E = 8
NCH_E = TPW // CH_E

_bf16 = jnp.bfloat16

_f32 = jnp.float32
_i32 = jnp.int32


def _dot_t(a, b):
    return lax.dot_general(a, b, (((1,), (1,)), ((), ())),
                           preferred_element_type=_f32)


def _silu(v):
    return v * (1.0 / (1.0 + jnp.exp(-v)))


def router_body(x_ref, rw_ref, i0_ref, i1_ref, w0_ref, w1_ref):
    l = _dot_t(rw_ref[...], x_ref[...])
    ei = lax.broadcasted_iota(_i32, (E, TB), 0)
    m0 = jnp.max(l, axis=0, keepdims=True)
    i0 = jnp.min(jnp.where(l == m0, ei, E), axis=0, keepdims=True)
    lm = jnp.where(ei == i0, -jnp.inf, l)
    m1 = jnp.max(lm, axis=0, keepdims=True)
    i1 = jnp.min(jnp.where(lm == m1, ei, E), axis=0, keepdims=True)
    w0 = 1.0 / (1.0 + jnp.exp(m1 - m0))
    w1 = 1.0 / (1.0 + jnp.exp(m0 - m1))
    i0_ref[...] = i0.reshape(1, 1, TB)
    i1_ref[...] = i1.reshape(1, 1, TB)
    w0_ref[...] = w0.reshape(1, 1, TB)
    w1_ref[...] = w1.reshape(1, 1, TB)


def _router_call(x2d, router_w):
    nb = N // TB
    o3 = jax.ShapeDtypeStruct((nb, 1, TB), _i32)
    o3f = jax.ShapeDtypeStruct((nb, 1, TB), _f32)
    return pl.pallas_call(
        router_body,
        grid=(nb,),
        in_specs=[
            pl.BlockSpec((TB, D), lambda b: (b, 0)),
            pl.BlockSpec((E, D), lambda b: (0, 0)),
        ],
        out_specs=[
            pl.BlockSpec((1, 1, TB), lambda b: (b, 0, 0)),
            pl.BlockSpec((1, 1, TB), lambda b: (b, 0, 0)),
            pl.BlockSpec((1, 1, TB), lambda b: (b, 0, 0)),
            pl.BlockSpec((1, 1, TB), lambda b: (b, 0, 0)),
        ],
        out_shape=[o3, o3, o3f, o3f],
    )(x2d, router_w)


def dispatch_body(i0_ref, i1_ref, d0_ref, d1_ref, be_ref):
    nb = N // TB
    i0 = i0_ref[...].reshape(nb, TB)
    i1 = i1_ref[...].reshape(nb, TB)

    ra = lax.broadcasted_iota(_i32, (TB, TB), 0)
    ca = lax.broadcasted_iota(_i32, (TB, TB), 1)
    U = (ra <= ca).astype(_f32)
    rb = lax.broadcasted_iota(_i32, (nb, nb), 0)
    cb = lax.broadcasted_iota(_i32, (nb, nb), 1)
    Ls = (rb > cb).astype(_f32)

    def excl_scan(m):
        rowcum = jnp.dot(m, U, preferred_element_type=_f32)
        rowtot = jnp.sum(m, axis=1, keepdims=True)
        rowoff = jnp.dot(Ls, rowtot, preferred_element_type=_f32)
        return rowcum - m + rowoff

    masks0 = [(i0 == e).astype(_f32) for e in range(E)]
    masks1 = [(i1 == e).astype(_f32) for e in range(E)]
    cnt0 = [jnp.sum(m).astype(_i32) for m in masks0]
    cnt1 = [jnp.sum(m).astype(_i32) for m in masks1]

    starts = []
    s = jnp.int32(0)
    for e in range(E):
        starts.append(s)
        seg = ((cnt0[e] + cnt1[e] + (BLK - 1)) // BLK) * BLK
        s = s + seg

    dest0 = jnp.zeros((nb, TB), _f32)
    dest1 = jnp.zeros((nb, TB), _f32)
    for e in range(E):
        r0 = excl_scan(masks0[e])
        r1 = excl_scan(masks1[e])
        base_e = starts[e].astype(_f32)
        dest0 = dest0 + masks0[e] * (base_e + r0)
        dest1 = dest1 + masks1[e] * (base_e + cnt0[e].astype(_f32) + r1)

    d0_ref[...] = dest0.astype(_i32).reshape(nb, 1, TB)
    d1_ref[...] = dest1.astype(_i32).reshape(nb, 1, TB)

    bvec = lax.broadcasted_iota(_i32, (8, 128), 1)
    acc = jnp.zeros((8, 128), _i32)
    for e in range(1, E):
        acc = acc + jnp.where(bvec >= starts[e] // BLK, 1, 0).astype(_i32)
    be_ref[...] = acc


def _dispatch_call(i0_3, i1_3):
    nb = N // TB
    return pl.pallas_call(
        dispatch_body,
        grid=(1,),
        in_specs=[
            pl.BlockSpec((nb, 1, TB), lambda i: (0, 0, 0)),
            pl.BlockSpec((nb, 1, TB), lambda i: (0, 0, 0)),
        ],
        out_specs=[
            pl.BlockSpec((nb, 1, TB), lambda i: (0, 0, 0)),
            pl.BlockSpec((nb, 1, TB), lambda i: (0, 0, 0)),
            pl.BlockSpec((8, 128), lambda i: (0, 0)),
        ],
        out_shape=[
            jax.ShapeDtypeStruct((nb, 1, TB), _i32),
            jax.ShapeDtypeStruct((nb, 1, TB), _i32),
            jax.ShapeDtypeStruct((8, 128), _i32),
        ],
    )(i0_3, i1_3)


@functools.lru_cache(maxsize=None)
def _scatter_dispatch_kernel():
    mesh = plsc.VectorSubcoreMesh(core_axis_name="c", subcore_axis_name="s")
    npair = NCH_C // 2

    @functools.partial(
        pl.kernel,
        out_type=(
            jax.ShapeDtypeStruct((NPAD, D), _f32),
            jax.ShapeDtypeStruct((NPAD,), _f32),
        ),
        mesh=mesh,
        scratch_types=[
            pltpu.VMEM((CH_C, D), _f32),
            pltpu.VMEM((CH_C, D), _f32),
            pltpu.VMEM((NCH_C, CH_C), _i32),
            pltpu.VMEM((NCH_C, CH_C), _i32),
            pltpu.VMEM((NCH_C, CH_C), _f32),
            pltpu.VMEM((NCH_C, CH_C), _f32),
            pltpu.SemaphoreType.DMA,
            pltpu.SemaphoreType.DMA,
            pltpu.SemaphoreType.DMA,
            pltpu.SemaphoreType.DMA,
        ],
    )
    def scatter_dispatch(x_hbm, d0_hbm, d1_hbm, w0_hbm, w1_hbm,
                         xg_hbm, ws_hbm, xvA, xvB, d0v, d1v, w0v, w1v,
                         semLA, semLB, semSA, semSB):
        wid = lax.axis_index("s") * NC + lax.axis_index("c")
        base = wid * TPW

        pltpu.sync_copy(d0_hbm.at[wid], d0v)
        pltpu.sync_copy(d1_hbm.at[wid], d1v)
        pltpu.sync_copy(w0_hbm.at[wid], w0v)
        pltpu.sync_copy(w1_hbm.at[wid], w1v)

        def load(c, xv, sem):
            pltpu.async_copy(x_hbm.at[pl.ds(base + c * CH_C, CH_C)], xv, sem)

        def wait_load(xv, sem):
            pltpu.make_async_copy(x_hbm.at[pl.ds(base, CH_C)], xv, sem).wait()

        def scatters(c, xv, sem):
            pltpu.async_copy(xv, xg_hbm.at[d0v.at[c]], sem)
            pltpu.async_copy(xv, xg_hbm.at[d1v.at[c]], sem)
            pltpu.async_copy(w0v.at[c], ws_hbm.at[d0v.at[c]], sem)
            pltpu.async_copy(w1v.at[c], ws_hbm.at[d1v.at[c]], sem)

        def wait_scatters(c, xv, sem):
            pltpu.make_async_copy(xv, xg_hbm.at[d0v.at[c]], sem).wait()
            pltpu.make_async_copy(xv, xg_hbm.at[d1v.at[c]], sem).wait()
            pltpu.make_async_copy(w0v.at[c], ws_hbm.at[d0v.at[c]], sem).wait()
            pltpu.make_async_copy(w1v.at[c], ws_hbm.at[d1v.at[c]], sem).wait()

        load(0, xvA, semLA)
        load(1, xvB, semLB)

        def pair(g, carry):
            a, b = 2 * g, 2 * g + 1
            wait_load(xvA, semLA)
            scatters(a, xvA, semSA)
            wait_load(xvB, semLB)
            scatters(b, xvB, semSB)
            wait_scatters(a, xvA, semSA)
            pl.when(g < npair - 1)(lambda: load(a + 2, xvA, semLA))
            wait_scatters(b, xvB, semSB)
            pl.when(g < npair - 1)(lambda: load(b + 2, xvB, semLB))
            return carry

        lax.fori_loop(0, npair, pair, 0)

    return scatter_dispatch


def group_gemm_body(be_ref, xg_ref, w1_ref, w2_ref, ws_ref, o_ref):
    h = _silu(_dot_t(xg_ref[...], w1_ref[0]))
    h = h * ws_ref[...]
    o_ref[...] = _dot_t(h, w2_ref[0])


def _group_gemm_call(be, xg, ew1, ew2, ws2):
    grid_spec = pltpu.PrefetchScalarGridSpec(
        num_scalar_prefetch=1,
        grid=(NBLK,),
        in_specs=[
            pl.BlockSpec((BLK, D), lambda b, be_r: (b, 0)),
            pl.BlockSpec((1, FF, D), lambda b, be_r: (be_r[b], 0, 0)),
            pl.BlockSpec((1, D, FF), lambda b, be_r: (be_r[b], 0, 0)),
            pl.BlockSpec((BLK, 1), lambda b, be_r: (b, 0)),
        ],
        out_specs=pl.BlockSpec((BLK, D), lambda b, be_r: (b, 0)),
    )
    return pl.pallas_call(
        group_gemm_body,
        grid_spec=grid_spec,
        out_shape=jax.ShapeDtypeStruct((NPAD, D), _f32),
    )(be, xg, ew1, ew2, ws2)


def shared_ffn_body(x_ref, w1_ref, w2_ref, o_ref):
    h = _silu(_dot_t(x_ref[...], w1_ref[...]))
    o_ref[...] = _dot_t(h, w2_ref[...])


def _shared_ffn_call(x2d, sw1, sw2):
    return pl.pallas_call(
        shared_ffn_body,
        grid=(N // BLK,),
        in_specs=[
            pl.BlockSpec((BLK, D), lambda b: (b, 0)),
            pl.BlockSpec((FF, D), lambda b: (0, 0)),
            pl.BlockSpec((D, FF), lambda b: (0, 0)),
        ],
        out_specs=pl.BlockSpec((BLK, D), lambda b: (b, 0)),
        out_shape=jax.ShapeDtypeStruct((N, D), _f32),
    )(x2d, sw1, sw2)


@functools.lru_cache(maxsize=None)
def _combine3_kernel():
    mesh = plsc.VectorSubcoreMesh(core_axis_name="c", subcore_axis_name="s")
    npair = NCH_E // 2

    @functools.partial(
        pl.kernel,
        out_type=jax.ShapeDtypeStruct((N, D), _f32),
        mesh=mesh,
        scratch_types=[
            pltpu.VMEM((NCH_E, CH_E), _i32),
            pltpu.VMEM((NCH_E, CH_E), _i32),
            pltpu.VMEM((CH_E, D), _f32),
            pltpu.VMEM((CH_E, D), _f32),
            pltpu.VMEM((CH_E, D), _f32),
            pltpu.VMEM((CH_E, D), _f32),
            pltpu.VMEM((CH_E, D), _f32),
            pltpu.VMEM((CH_E, D), _f32),
            pltpu.SemaphoreType.DMA,
            pltpu.SemaphoreType.DMA,
            pltpu.SemaphoreType.DMA,
            pltpu.SemaphoreType.DMA,
        ],
    )
    def combine3(rows_hbm, sh_hbm, d0_hbm, d1_hbm, out_hbm,
                 p0v, p1v, accA, r1A, shA, accB, r1B, shB,
                 semA, semB, semStA, semStB):
        wid = lax.axis_index("s") * NC + lax.axis_index("c")
        base = wid * TPW

        pltpu.sync_copy(d0_hbm.at[wid], p0v)
        pltpu.sync_copy(d1_hbm.at[wid], p1v)

        def gathers(c, acc, r1, sh, sem):
            pltpu.async_copy(rows_hbm.at[p0v.at[c]], acc, sem)
            pltpu.async_copy(rows_hbm.at[p1v.at[c]], r1, sem)
            pltpu.async_copy(sh_hbm.at[pl.ds(base + c * CH_E, CH_E)], sh, sem)

        def wait_gathers(c, acc, r1, sh, sem):
            pltpu.make_async_copy(rows_hbm.at[p0v.at[c]], acc, sem).wait()
            pltpu.make_async_copy(rows_hbm.at[p1v.at[c]], r1, sem).wait()
            pltpu.make_async_copy(
                sh_hbm.at[pl.ds(base, CH_E)], sh, sem).wait()

        def compute(acc, r1, sh):
            for r in range(CH_E):
                def col(j, jc):
                    sl = pl.ds(j * 16, 16)
                    acc[r, sl] = acc[r, sl] + r1[r, sl] + sh[r, sl]
                    return jc
                lax.fori_loop(0, D // 16, col, 0, unroll=8)

        def store(c, acc, sem):
            pltpu.async_copy(acc, out_hbm.at[pl.ds(base + c * CH_E, CH_E)], sem)

        def wait_store(acc, sem):
            pltpu.make_async_copy(
                acc, out_hbm.at[pl.ds(base, CH_E)], sem).wait()

        gathers(0, accA, r1A, shA, semA)

        def pair(g, carry):
            a, b = 2 * g, 2 * g + 1
            wait_gathers(a, accA, r1A, shA, semA)
            gathers(b, accB, r1B, shB, semB)
            compute(accA, r1A, shA)
            store(a, accA, semStA)
            wait_gathers(b, accB, r1B, shB, semB)
            wait_store(accA, semStA)
            pl.when(g < npair - 1)(
                lambda: gathers(a + 2, accA, r1A, shA, semA))
            compute(accB, r1B, shB)
            store(b, accB, semStB)
            wait_store(accB, semStB)
            return carry

        lax.fori_loop(0, npair, pair, 0)

    return combine3


def kernel(x, shared_w1, shared_w2, expert_w1, expert_w2, router_w):
    bs, seq, d = x.shape
    x2d = x.reshape(N, D)

    i0_3, i1_3, w0_3, w1_3 = _router_call(x2d, router_w)
    d0_3, d1_3, be_full = _dispatch_call(i0_3, i1_3)

    d0c = d0_3.reshape(NW, NCH_C, CH_C)
    d1c = d1_3.reshape(NW, NCH_C, CH_C)
    w0c = w0_3.reshape(NW, NCH_C, CH_C)
    w1c = w1_3.reshape(NW, NCH_C, CH_C)
    d0e = d0_3.reshape(NW, NCH_E, CH_E)
    d1e = d1_3.reshape(NW, NCH_E, CH_E)
    be = be_full[0, :NBLK]

    sh = _shared_ffn_call(x2d, shared_w1[0], shared_w2[0])
    xg, ws = _scatter_dispatch_kernel()(x2d, d0c, d1c, w0c, w1c)
    rows = _group_gemm_call(be, xg, expert_w1, expert_w2, ws.reshape(NPAD, 1))
    out2d = _combine3_kernel()(rows, sh, d0e, d1e)

    return (out2d.reshape(bs, seq, d), jnp.float32(0.0))
